# speedup vs baseline: 1.5277x; 1.5277x over previous
import jax
import jax.numpy as jnp
from jax.experimental import pallas as pl
from jax.experimental.pallas import tpu as pltpu

_B, _C, _H, _W = 4, 96, 512, 512
_BH = 32
_TOTAL = _B * _H * _W
_NF = 1.0 / _C


def _body(x_ref, loss_ref, dist_ref, acc_ref):
    step = pl.program_id(0) * pl.num_programs(1) + pl.program_id(1)
    nsteps = pl.num_programs(0) * pl.num_programs(1)

    @pl.when(step == 0)
    def _init():
        acc_ref[...] = jnp.zeros_like(acc_ref)

    x = x_ref[0]
    idx = jnp.argmax(x, axis=0).astype(jnp.int32)
    classes = jax.lax.broadcasted_iota(jnp.int32, (_C, _BH, _W), 0)
    onehot = (idx[None, :, :] == classes).astype(jnp.float32)
    acc_ref[...] += jnp.sum(onehot, axis=1)

    @pl.when(step == nsteps - 1)
    def _fin():
        hist = jnp.sum(acc_ref[...], axis=1, keepdims=True)
        dist = hist * (1.0 / _TOTAL)
        dist_ref[...] = dist
        z = (dist - _NF) * (1.0 / (1.0 - _NF))
        loss_ref[0, 0] = jnp.sqrt(jnp.sum(z * z))


def kernel(generated_masks):
    loss2d, dist2d = pl.pallas_call(
        _body,
        grid=(_B, _H // _BH),
        in_specs=[
            pl.BlockSpec((1, _C, _BH, _W), lambda b, h: (b, 0, h, 0)),
        ],
        out_specs=[
            pl.BlockSpec(memory_space=pltpu.SMEM),
            pl.BlockSpec((_C, 1), lambda b, h: (0, 0)),
        ],
        out_shape=[
            jax.ShapeDtypeStruct((1, 1), jnp.float32),
            jax.ShapeDtypeStruct((_C, 1), jnp.float32),
        ],
        scratch_shapes=[pltpu.VMEM((_C, _W), jnp.float32)],
    )(generated_masks)
    return (loss2d[0, 0], dist2d[:, 0])
